# Initial kernel scaffold; baseline (speedup 1.0000x reference)
import jax
import jax.numpy as jnp
from jax.experimental import pallas as pl


def kernel(X, edge_index, L_values, batch, W1, b1, W2, b2, W3, b3, Wl, bl):
    raise NotImplementedError("write your pallas kernel here")


# speedup vs baseline: 5.9044x; 5.9044x over previous
import functools

import jax
import jax.numpy as jnp
from jax import lax
from jax.experimental import pallas as pl
from jax.experimental.pallas import tpu as pltpu
from jax.experimental.pallas import tpu_sc as plsc

N = 10000
NPAD = 10112
E = 320000
IN = 128
F = 64
OUT = 10
G = 64

EBLK = 128
NC, NS = 2, 16
NW = NC * NS
NBLK = 2560
BPW = NBLK // NW
EPAD = NBLK * EBLK
RSUB = NPAD // NS

_MESH = plsc.VectorSubcoreMesh(core_axis_name="c", subcore_axis_name="s")


def _edge_pass(width, weighted, with_deg):
    out_types = [jax.ShapeDtypeStruct((NC, NPAD, width), jnp.float32)]
    if with_deg:
        out_types.append(jax.ShapeDtypeStruct((NC, NPAD, 16), jnp.float32))

    scratch = [
        pltpu.VMEM((EBLK,), jnp.int32),
        pltpu.VMEM((EBLK,), jnp.int32),
        pltpu.VMEM((EBLK, width), jnp.float32),
        pltpu.VMEM_SHARED((NPAD, width), jnp.float32),
        pltpu.SemaphoreType.DMA,
    ]
    if weighted:
        scratch.append(pltpu.VMEM((EBLK,), jnp.float32))
    if with_deg:
        scratch.append(pltpu.VMEM((EBLK, 16), jnp.float32))
        scratch.append(pltpu.VMEM_SHARED((NPAD, 16), jnp.float32))

    def body(*refs):
        i = 0
        table_h = refs[i]; i += 1
        gidx_h = refs[i]; i += 1
        sidx_h = refs[i]; i += 1
        if weighted:
            lv_h = refs[i]; i += 1
        zeros_h = refs[i]; i += 1
        if with_deg:
            zeros16_h = refs[i]; i += 1
            ones_h = refs[i]; i += 1
        out_h = refs[i]; i += 1
        if with_deg:
            dego_h = refs[i]; i += 1
        gidx_v = refs[i]; i += 1
        sidx_v = refs[i]; i += 1
        rows_v = refs[i]; i += 1
        acc = refs[i]; i += 1
        sem = refs[i]; i += 1
        if weighted:
            lv_v = refs[i]; i += 1
        if with_deg:
            ones_v = refs[i]; i += 1
            dacc = refs[i]; i += 1

        c = lax.axis_index("c")
        s = lax.axis_index("s")
        wid = s * NC + c
        r0 = s * RSUB

        pltpu.sync_copy(zeros_h.at[pl.ds(r0, RSUB), :],
                        acc.at[pl.ds(r0, RSUB), :])
        if with_deg:
            pltpu.sync_copy(zeros16_h.at[pl.ds(r0, RSUB), :],
                            dacc.at[pl.ds(r0, RSUB), :])
            pltpu.sync_copy(ones_h, ones_v)
        plsc.subcore_barrier()

        @pl.loop(0, BPW)
        def _(ib):
            e0 = (wid * BPW + ib) * EBLK
            pltpu.sync_copy(gidx_h.at[pl.ds(e0, EBLK)], gidx_v)
            pltpu.sync_copy(sidx_h.at[pl.ds(e0, EBLK)], sidx_v)
            if weighted:
                pltpu.sync_copy(lv_h.at[pl.ds(e0, EBLK)], lv_v)
            pltpu.async_copy(table_h.at[gidx_v], rows_v, sem).wait()
            if weighted:
                @pl.loop(0, EBLK // 16)
                def _(eg):
                    vals = lv_v[pl.ds(eg * 16, 16)]
                    for j in range(16):
                        val = vals[j]
                        e = eg * 16 + j
                        for k in range(width // 16):
                            sl = (e, pl.ds(k * 16, 16))
                            rows_v[sl] = rows_v[sl] * val
            pltpu.sync_copy(rows_v, acc.at[sidx_v], add=True)
            if with_deg:
                pltpu.sync_copy(ones_v, dacc.at[gidx_v], add=True)

        plsc.subcore_barrier()
        pltpu.sync_copy(acc.at[pl.ds(r0, RSUB), :],
                        out_h.at[c, pl.ds(r0, RSUB), :])
        if with_deg:
            pltpu.sync_copy(dacc.at[pl.ds(r0, RSUB), :],
                            dego_h.at[c, pl.ds(r0, RSUB), :])

    return functools.partial(
        pl.kernel, mesh=_MESH, out_type=out_types, scratch_types=scratch,
        compiler_params=pltpu.CompilerParams(use_tc_tiling_on_sc=False),
    )(body)


_pass_uv = _edge_pass(2 * F, weighted=True, with_deg=True)
_pass_w64 = _edge_pass(F, weighted=True, with_deg=False)
_pass_u64 = _edge_pass(F, weighted=False, with_deg=False)


def _tc_call(fn, out_shapes):
    return pl.pallas_call(fn, out_shape=out_shapes)


def _tc_a(x_ref, w1_ref, uv_ref, w_ref):
    x = x_ref[...]
    w1 = w1_ref[...]
    a, b, c = w1[0:IN], w1[IN:2 * IN], w1[2 * IN:3 * IN]
    uv_ref[:, 0:F] = jnp.dot(x, b, preferred_element_type=jnp.float32)
    uv_ref[:, F:2 * F] = jnp.dot(x, c, preferred_element_type=jnp.float32)
    w_ref[...] = jnp.dot(x, a - c, preferred_element_type=jnp.float32)


def _tc_b(uvp_ref, degp_ref, su_ref, sv_ref, dinv_ref):
    su_ref[...] = uvp_ref[0, :, 0:F] + uvp_ref[1, :, 0:F]
    sv_ref[...] = uvp_ref[0, :, F:2 * F] + uvp_ref[1, :, F:2 * F]
    deg = degp_ref[0, :, 0:1] + degp_ref[1, :, 0:1] + 1.0
    dinv_ref[...] = lax.rsqrt(deg)


def _tc_c(w_ref, su_ref, ssvp_ref, dinv_ref, h1_ref, q1_ref):
    h1 = w_ref[...] + su_ref[...] + 2.0 * (ssvp_ref[0] + ssvp_ref[1])
    h1_ref[...] = h1
    q1_ref[...] = dinv_ref[...] * h1


def _tc_conv(mp_ref, h_ref, dinv_ref, b_ref, w_ref, x_ref, hn_ref, qn_ref):
    dinv = dinv_ref[...]
    m = mp_ref[0] + mp_ref[1]
    xk = jax.nn.relu(dinv * m + dinv * dinv * h_ref[...] + b_ref[...])
    rowid = lax.broadcasted_iota(jnp.int32, (NPAD, F), 0)
    xk = jnp.where(rowid < N, xk, 0.0)
    x_ref[...] = xk
    hn = jnp.dot(xk, w_ref[...], preferred_element_type=jnp.float32)
    hn_ref[...] = hn
    qn_ref[...] = dinv * hn


def _tc_f(mp_ref, h3_ref, dinv_ref, b3_ref, x1_ref, x2_ref, batch_ref,
          wl_ref, bl_ref, out_ref):
    dinv = dinv_ref[...]
    m = mp_ref[0] + mp_ref[1]
    x3 = jax.nn.relu(dinv * m + dinv * dinv * h3_ref[...] + b3_ref[...])
    xm = (x1_ref[...] + x2_ref[...] + x3) * (1.0 / 3.0)
    gid = lax.broadcasted_iota(jnp.int32, (G, NPAD), 0)
    onehot = (batch_ref[...][None, :] == gid).astype(jnp.float32)
    sums = jnp.dot(onehot, xm, preferred_element_type=jnp.float32)
    cnt = jnp.sum(onehot, axis=1, keepdims=True)
    pooled = sums / jnp.maximum(cnt, 1.0)
    logits = jnp.dot(pooled, wl_ref[...],
                     preferred_element_type=jnp.float32) + bl_ref[...]
    z = logits - jnp.max(logits, axis=1, keepdims=True)
    ez = jnp.exp(z)
    out_ref[...] = ez / jnp.sum(ez, axis=1, keepdims=True)


def kernel(X, edge_index, L_values, batch, W1, b1, W2, b2, W3, b3, Wl, bl):
    f32 = jnp.float32
    x0 = X[0].astype(f32)
    ei = edge_index.astype(jnp.int32)
    row, col = ei[0], ei[1]
    b0 = batch[0].astype(jnp.int32)

    pad = EPAD - E
    padi = jnp.full((pad,), N, jnp.int32)
    rowp = jnp.concatenate([row, padi])
    colp = jnp.concatenate([col, padi])
    lvp = jnp.concatenate([L_values.astype(f32), jnp.zeros((pad,), f32)])

    x0p = jnp.zeros((NPAD, IN), f32).at[:N].set(x0)
    b0p = jnp.concatenate([b0, jnp.full((NPAD - N,), G, jnp.int32)])
    zeros128 = jnp.zeros((NPAD, 2 * F), f32)
    zeros64 = jnp.zeros((NPAD, F), f32)
    zeros16 = jnp.zeros((NPAD, 16), f32)
    ones16 = jnp.ones((EBLK, 16), f32)

    uv, w = _tc_call(_tc_a, [jax.ShapeDtypeStruct((NPAD, 2 * F), f32),
                             jax.ShapeDtypeStruct((NPAD, F), f32)])(x0p, W1)

    uvp, degp = _pass_uv(uv, colp, rowp, lvp, zeros128, zeros16, ones16)

    su, sv, dinv = _tc_call(_tc_b, [jax.ShapeDtypeStruct((NPAD, F), f32),
                                    jax.ShapeDtypeStruct((NPAD, F), f32),
                                    jax.ShapeDtypeStruct((NPAD, 1), f32)])(
        uvp, degp)

    (ssvp,) = _pass_w64(sv, colp, rowp, lvp, zeros64)

    h1, q1 = _tc_call(_tc_c, [jax.ShapeDtypeStruct((NPAD, F), f32),
                              jax.ShapeDtypeStruct((NPAD, F), f32)])(
        w, su, ssvp, dinv)

    (m1p,) = _pass_u64(q1, rowp, colp, zeros64)
    x1, h2, q2 = _tc_call(_tc_conv, [jax.ShapeDtypeStruct((NPAD, F), f32)] * 3)(
        m1p, h1, dinv, b1, W2)
    (m2p,) = _pass_u64(q2, rowp, colp, zeros64)
    x2, h3, q3 = _tc_call(_tc_conv, [jax.ShapeDtypeStruct((NPAD, F), f32)] * 3)(
        m2p, h2, dinv, b2, W3)
    (m3p,) = _pass_u64(q3, rowp, colp, zeros64)

    out = _tc_call(_tc_f, jax.ShapeDtypeStruct((G, OUT), f32))(
        m3p, h3, dinv, b3, x1, x2, b0p, Wl, bl)
    return out


# speedup vs baseline: 7.9370x; 1.3443x over previous
import functools

import jax
import jax.numpy as jnp
from jax import lax
from jax.experimental import pallas as pl
from jax.experimental.pallas import tpu as pltpu
from jax.experimental.pallas import tpu_sc as plsc

N = 10000
NPAD = 10112
E = 320000
IN = 128
F = 64
OUT = 10
G = 64

EBLK = 128
NC, NS = 2, 16
NW = NC * NS
NBLK = 2560
BPW = NBLK // NW
EPAD = NBLK * EBLK
RSUB = NPAD // NS

_MESH = plsc.VectorSubcoreMesh(core_axis_name="c", subcore_axis_name="s")


def _edge_pass(width, weighted, with_deg):
    out_types = [jax.ShapeDtypeStruct((NC, NPAD, width), jnp.float32)]
    if with_deg:
        out_types.append(jax.ShapeDtypeStruct((NC, NPAD, 16), jnp.float32))

    scratch = [
        pltpu.VMEM((2, EBLK), jnp.int32),
        pltpu.VMEM((2, EBLK), jnp.int32),
        pltpu.VMEM((2, EBLK, width), jnp.float32),
        pltpu.VMEM_SHARED((NPAD, width), jnp.float32),
        pltpu.SemaphoreType.DMA((2,)),
        pltpu.SemaphoreType.DMA((2,)),
    ]
    if weighted:
        scratch.append(pltpu.VMEM((2, EBLK), jnp.float32))
    if with_deg:
        scratch.append(pltpu.VMEM((EBLK, 16), jnp.float32))
        scratch.append(pltpu.VMEM_SHARED((NPAD, 16), jnp.float32))

    def body(*refs):
        i = 0
        table_h = refs[i]; i += 1
        gidx_h = refs[i]; i += 1
        sidx_h = refs[i]; i += 1
        if weighted:
            lv_h = refs[i]; i += 1
        zeros_h = refs[i]; i += 1
        if with_deg:
            zeros16_h = refs[i]; i += 1
            ones_h = refs[i]; i += 1
        out_h = refs[i]; i += 1
        if with_deg:
            dego_h = refs[i]; i += 1
        gidx_v = refs[i]; i += 1
        sidx_v = refs[i]; i += 1
        rows_v = refs[i]; i += 1
        acc = refs[i]; i += 1
        sem_i = refs[i]; i += 1
        sem_g = refs[i]; i += 1
        if weighted:
            lv_v = refs[i]; i += 1
        if with_deg:
            ones_v = refs[i]; i += 1
            dacc = refs[i]; i += 1

        c = lax.axis_index("c")
        s = lax.axis_index("s")
        wid = s * NC + c
        r0 = s * RSUB
        base = wid * BPW

        def issue_idx(ib, k):
            e0 = (base + ib) * EBLK
            pltpu.async_copy(gidx_h.at[pl.ds(e0, EBLK)], gidx_v.at[k],
                             sem_i.at[k])
            pltpu.async_copy(sidx_h.at[pl.ds(e0, EBLK)], sidx_v.at[k],
                             sem_i.at[k])
            if weighted:
                pltpu.async_copy(lv_h.at[pl.ds(e0, EBLK)], lv_v.at[k],
                                 sem_i.at[k])

        def wait_idx(ib, k):
            e0 = (base + ib) * EBLK
            pltpu.make_async_copy(gidx_h.at[pl.ds(e0, EBLK)], gidx_v.at[k],
                                  sem_i.at[k]).wait()
            pltpu.make_async_copy(sidx_h.at[pl.ds(e0, EBLK)], sidx_v.at[k],
                                  sem_i.at[k]).wait()
            if weighted:
                pltpu.make_async_copy(lv_h.at[pl.ds(e0, EBLK)], lv_v.at[k],
                                      sem_i.at[k]).wait()

        def issue_gather(k):
            pltpu.async_copy(table_h.at[gidx_v.at[k]], rows_v.at[k],
                             sem_g.at[k])

        def wait_gather(k):
            pltpu.make_async_copy(table_h.at[gidx_v.at[k]], rows_v.at[k],
                                  sem_g.at[k]).wait()

        pltpu.sync_copy(zeros_h.at[pl.ds(r0, RSUB), :],
                        acc.at[pl.ds(r0, RSUB), :])
        if with_deg:
            pltpu.sync_copy(zeros16_h.at[pl.ds(r0, RSUB), :],
                            dacc.at[pl.ds(r0, RSUB), :])
            pltpu.sync_copy(ones_h, ones_v)
        plsc.subcore_barrier()

        issue_idx(0, 0)
        wait_idx(0, 0)
        issue_gather(0)

        @pl.loop(0, BPW // 2)
        def _(g):
            for k in (0, 1):
                ib = g * 2 + k

                @pl.when(ib + 1 < BPW)
                def _():
                    issue_idx(ib + 1, 1 - k)
                wait_gather(k)

                @pl.when(ib + 1 < BPW)
                def _():
                    wait_idx(ib + 1, 1 - k)
                    issue_gather(1 - k)
                if weighted:
                    @pl.loop(0, EBLK // 16)
                    def _(eg):
                        vals = lv_v[k, pl.ds(eg * 16, 16)]
                        for j in range(16):
                            val = vals[j]
                            e = eg * 16 + j
                            for q in range(width // 16):
                                sl = (k, e, pl.ds(q * 16, 16))
                                rows_v[sl] = rows_v[sl] * val
                pltpu.sync_copy(rows_v.at[k], acc.at[sidx_v.at[k]], add=True)
                if with_deg:
                    pltpu.sync_copy(ones_v, dacc.at[gidx_v.at[k]], add=True)

        plsc.subcore_barrier()
        pltpu.sync_copy(acc.at[pl.ds(r0, RSUB), :],
                        out_h.at[c, pl.ds(r0, RSUB), :])
        if with_deg:
            pltpu.sync_copy(dacc.at[pl.ds(r0, RSUB), :],
                            dego_h.at[c, pl.ds(r0, RSUB), :])

    return functools.partial(
        pl.kernel, mesh=_MESH, out_type=out_types, scratch_types=scratch,
        compiler_params=pltpu.CompilerParams(use_tc_tiling_on_sc=False),
    )(body)


_pass_uv = _edge_pass(2 * F, weighted=True, with_deg=True)
_pass_w64 = _edge_pass(F, weighted=True, with_deg=False)
_pass_u64 = _edge_pass(F, weighted=False, with_deg=False)


def _tc_call(fn, out_shapes):
    return pl.pallas_call(fn, out_shape=out_shapes)


def _tc_a(x_ref, w1_ref, uv_ref, w_ref):
    x = x_ref[...]
    w1 = w1_ref[...]
    a, b, c = w1[0:IN], w1[IN:2 * IN], w1[2 * IN:3 * IN]
    uv_ref[:, 0:F] = jnp.dot(x, b, preferred_element_type=jnp.float32)
    uv_ref[:, F:2 * F] = jnp.dot(x, c, preferred_element_type=jnp.float32)
    w_ref[...] = jnp.dot(x, a - c, preferred_element_type=jnp.float32)


def _tc_b(uvp_ref, degp_ref, su_ref, sv_ref, dinv_ref):
    su_ref[...] = uvp_ref[0, :, 0:F] + uvp_ref[1, :, 0:F]
    sv_ref[...] = uvp_ref[0, :, F:2 * F] + uvp_ref[1, :, F:2 * F]
    deg = degp_ref[0, :, 0:1] + degp_ref[1, :, 0:1] + 1.0
    dinv_ref[...] = lax.rsqrt(deg)


def _tc_c(w_ref, su_ref, ssvp_ref, dinv_ref, h1_ref, q1_ref):
    h1 = w_ref[...] + su_ref[...] + 2.0 * (ssvp_ref[0] + ssvp_ref[1])
    h1_ref[...] = h1
    q1_ref[...] = dinv_ref[...] * h1


def _tc_conv(mp_ref, h_ref, dinv_ref, b_ref, w_ref, x_ref, hn_ref, qn_ref):
    dinv = dinv_ref[...]
    m = mp_ref[0] + mp_ref[1]
    xk = jax.nn.relu(dinv * m + dinv * dinv * h_ref[...] + b_ref[...])
    rowid = lax.broadcasted_iota(jnp.int32, (NPAD, F), 0)
    xk = jnp.where(rowid < N, xk, 0.0)
    x_ref[...] = xk
    hn = jnp.dot(xk, w_ref[...], preferred_element_type=jnp.float32)
    hn_ref[...] = hn
    qn_ref[...] = dinv * hn


def _tc_f(mp_ref, h3_ref, dinv_ref, b3_ref, x1_ref, x2_ref, batch_ref,
          wl_ref, bl_ref, out_ref):
    dinv = dinv_ref[...]
    m = mp_ref[0] + mp_ref[1]
    x3 = jax.nn.relu(dinv * m + dinv * dinv * h3_ref[...] + b3_ref[...])
    xm = (x1_ref[...] + x2_ref[...] + x3) * (1.0 / 3.0)
    gid = lax.broadcasted_iota(jnp.int32, (G, NPAD), 0)
    onehot = (batch_ref[...][None, :] == gid).astype(jnp.float32)
    sums = jnp.dot(onehot, xm, preferred_element_type=jnp.float32)
    cnt = jnp.sum(onehot, axis=1, keepdims=True)
    pooled = sums / jnp.maximum(cnt, 1.0)
    logits = jnp.dot(pooled, wl_ref[...],
                     preferred_element_type=jnp.float32) + bl_ref[...]
    z = logits - jnp.max(logits, axis=1, keepdims=True)
    ez = jnp.exp(z)
    out_ref[...] = ez / jnp.sum(ez, axis=1, keepdims=True)


def kernel(X, edge_index, L_values, batch, W1, b1, W2, b2, W3, b3, Wl, bl):
    f32 = jnp.float32
    x0 = X[0].astype(f32)
    ei = edge_index.astype(jnp.int32)
    row, col = ei[0], ei[1]
    b0 = batch[0].astype(jnp.int32)

    pad = EPAD - E
    padi = jnp.full((pad,), N, jnp.int32)
    rowp = jnp.concatenate([row, padi])
    colp = jnp.concatenate([col, padi])
    lvp = jnp.concatenate([L_values.astype(f32), jnp.zeros((pad,), f32)])

    x0p = jnp.zeros((NPAD, IN), f32).at[:N].set(x0)
    b0p = jnp.concatenate([b0, jnp.full((NPAD - N,), G, jnp.int32)])
    zeros128 = jnp.zeros((NPAD, 2 * F), f32)
    zeros64 = jnp.zeros((NPAD, F), f32)
    zeros16 = jnp.zeros((NPAD, 16), f32)
    ones16 = jnp.ones((EBLK, 16), f32)

    uv, w = _tc_call(_tc_a, [jax.ShapeDtypeStruct((NPAD, 2 * F), f32),
                             jax.ShapeDtypeStruct((NPAD, F), f32)])(x0p, W1)

    uvp, degp = _pass_uv(uv, colp, rowp, lvp, zeros128, zeros16, ones16)

    su, sv, dinv = _tc_call(_tc_b, [jax.ShapeDtypeStruct((NPAD, F), f32),
                                    jax.ShapeDtypeStruct((NPAD, F), f32),
                                    jax.ShapeDtypeStruct((NPAD, 1), f32)])(
        uvp, degp)

    (ssvp,) = _pass_w64(sv, colp, rowp, lvp, zeros64)

    h1, q1 = _tc_call(_tc_c, [jax.ShapeDtypeStruct((NPAD, F), f32),
                              jax.ShapeDtypeStruct((NPAD, F), f32)])(
        w, su, ssvp, dinv)

    (m1p,) = _pass_u64(q1, rowp, colp, zeros64)
    x1, h2, q2 = _tc_call(_tc_conv, [jax.ShapeDtypeStruct((NPAD, F), f32)] * 3)(
        m1p, h1, dinv, b1, W2)
    (m2p,) = _pass_u64(q2, rowp, colp, zeros64)
    x2, h3, q3 = _tc_call(_tc_conv, [jax.ShapeDtypeStruct((NPAD, F), f32)] * 3)(
        m2p, h2, dinv, b2, W3)
    (m3p,) = _pass_u64(q3, rowp, colp, zeros64)

    out = _tc_call(_tc_f, jax.ShapeDtypeStruct((G, OUT), f32))(
        m3p, h3, dinv, b3, x1, x2, b0p, Wl, bl)
    return out


# speedup vs baseline: 8.8762x; 1.1183x over previous
import functools

import jax
import jax.numpy as jnp
from jax import lax
from jax.experimental import pallas as pl
from jax.experimental.pallas import tpu as pltpu
from jax.experimental.pallas import tpu_sc as plsc

N = 10000
NPAD = 10112
E = 320000
IN = 128
F = 64
OUT = 10
G = 64

EBLK = 128
NC, NS = 2, 16
NW = NC * NS
NBLK = 2560
EPAD = NBLK * EBLK
RSUB = NPAD // NS
BPW0 = 116
BPW1 = (NBLK - BPW0 * NS) // NS
assert BPW0 * NS + BPW1 * NS == NBLK and BPW0 % 2 == 0 and BPW1 % 2 == 0

_MESH = plsc.VectorSubcoreMesh(core_axis_name="c", subcore_axis_name="s")


def _edge_pass(width, weighted, with_deg):
    out_types = [jax.ShapeDtypeStruct((NC, NPAD, width), jnp.float32)]
    if with_deg:
        out_types.append(jax.ShapeDtypeStruct((NC, NPAD, 16), jnp.float32))

    scratch = [
        pltpu.VMEM((2, EBLK), jnp.int32),
        pltpu.VMEM((2, EBLK), jnp.int32),
        pltpu.VMEM((2, EBLK, width), jnp.float32),
        pltpu.VMEM_SHARED((NPAD, width), jnp.float32),
        pltpu.SemaphoreType.DMA((2,)),
        pltpu.SemaphoreType.DMA((2,)),
    ]
    if weighted:
        scratch.append(pltpu.VMEM((2, EBLK), jnp.float32))
    if with_deg:
        scratch.append(pltpu.VMEM((EBLK, 16), jnp.float32))
        scratch.append(pltpu.VMEM_SHARED((NPAD, 16), jnp.float32))

    def body(*refs):
        i = 0
        table_h = refs[i]; i += 1
        gidx_h = refs[i]; i += 1
        sidx_h = refs[i]; i += 1
        if weighted:
            lv_h = refs[i]; i += 1
        zeros_h = refs[i]; i += 1
        if with_deg:
            zeros16_h = refs[i]; i += 1
            ones_h = refs[i]; i += 1
        out_h = refs[i]; i += 1
        if with_deg:
            dego_h = refs[i]; i += 1
        gidx_v = refs[i]; i += 1
        sidx_v = refs[i]; i += 1
        rows_v = refs[i]; i += 1
        acc = refs[i]; i += 1
        sem_i = refs[i]; i += 1
        sem_g = refs[i]; i += 1
        if weighted:
            lv_v = refs[i]; i += 1
        if with_deg:
            ones_v = refs[i]; i += 1
            dacc = refs[i]; i += 1

        c = lax.axis_index("c")
        s = lax.axis_index("s")
        r0 = s * RSUB
        bpw = lax.select(c == 0, jnp.int32(BPW0), jnp.int32(BPW1))
        base = lax.select(c == 0, s * BPW0, NS * BPW0 + s * BPW1)

        def issue_idx(ib, k):
            e0 = (base + ib) * EBLK
            pltpu.async_copy(gidx_h.at[pl.ds(e0, EBLK)], gidx_v.at[k],
                             sem_i.at[k])
            pltpu.async_copy(sidx_h.at[pl.ds(e0, EBLK)], sidx_v.at[k],
                             sem_i.at[k])
            if weighted:
                pltpu.async_copy(lv_h.at[pl.ds(e0, EBLK)], lv_v.at[k],
                                 sem_i.at[k])

        def wait_idx(ib, k):
            e0 = (base + ib) * EBLK
            pltpu.make_async_copy(gidx_h.at[pl.ds(e0, EBLK)], gidx_v.at[k],
                                  sem_i.at[k]).wait()
            pltpu.make_async_copy(sidx_h.at[pl.ds(e0, EBLK)], sidx_v.at[k],
                                  sem_i.at[k]).wait()
            if weighted:
                pltpu.make_async_copy(lv_h.at[pl.ds(e0, EBLK)], lv_v.at[k],
                                      sem_i.at[k]).wait()

        def issue_gather(k):
            pltpu.async_copy(table_h.at[gidx_v.at[k]], rows_v.at[k],
                             sem_g.at[k])

        def wait_gather(k):
            pltpu.make_async_copy(table_h.at[gidx_v.at[k]], rows_v.at[k],
                                  sem_g.at[k]).wait()

        pltpu.sync_copy(zeros_h.at[pl.ds(r0, RSUB), :],
                        acc.at[pl.ds(r0, RSUB), :])
        if with_deg:
            pltpu.sync_copy(zeros16_h.at[pl.ds(r0, RSUB), :],
                            dacc.at[pl.ds(r0, RSUB), :])
            pltpu.sync_copy(ones_h, ones_v)
        plsc.subcore_barrier()

        issue_idx(0, 0)
        wait_idx(0, 0)
        issue_gather(0)

        @pl.loop(0, bpw // 2)
        def _(g):
            for k in (0, 1):
                ib = g * 2 + k

                @pl.when(ib + 1 < bpw)
                def _():
                    issue_idx(ib + 1, 1 - k)
                wait_gather(k)

                @pl.when(ib + 1 < bpw)
                def _():
                    wait_idx(ib + 1, 1 - k)
                    issue_gather(1 - k)
                if weighted:
                    @pl.loop(0, EBLK // 16)
                    def _(eg):
                        vals = lv_v[k, pl.ds(eg * 16, 16)]
                        for j in range(16):
                            val = vals[j]
                            e = eg * 16 + j
                            for q in range(width // 16):
                                sl = (k, e, pl.ds(q * 16, 16))
                                rows_v[sl] = rows_v[sl] * val
                pltpu.sync_copy(rows_v.at[k], acc.at[sidx_v.at[k]], add=True)
                if with_deg:
                    pltpu.sync_copy(ones_v, dacc.at[gidx_v.at[k]], add=True)

        plsc.subcore_barrier()
        pltpu.sync_copy(acc.at[pl.ds(r0, RSUB), :],
                        out_h.at[c, pl.ds(r0, RSUB), :])
        if with_deg:
            pltpu.sync_copy(dacc.at[pl.ds(r0, RSUB), :],
                            dego_h.at[c, pl.ds(r0, RSUB), :])

    return functools.partial(
        pl.kernel, mesh=_MESH, out_type=out_types, scratch_types=scratch,
        compiler_params=pltpu.CompilerParams(use_tc_tiling_on_sc=False),
    )(body)


_pass_uv = _edge_pass(2 * F, weighted=True, with_deg=True)
_pass_w64 = _edge_pass(F, weighted=True, with_deg=False)
_pass_u64 = _edge_pass(F, weighted=False, with_deg=False)


def _tc_call(fn, out_shapes):
    return pl.pallas_call(fn, out_shape=out_shapes)


def _tc_a(x_ref, w1_ref, uv_ref, w_ref):
    x = x_ref[...]
    w1 = w1_ref[...]
    a, b, c = w1[0:IN], w1[IN:2 * IN], w1[2 * IN:3 * IN]
    uv_ref[:, 0:F] = jnp.dot(x, b, preferred_element_type=jnp.float32)
    uv_ref[:, F:2 * F] = jnp.dot(x, c, preferred_element_type=jnp.float32)
    w_ref[...] = jnp.dot(x, a - c, preferred_element_type=jnp.float32)


def _tc_b(uvp_ref, degp_ref, su_ref, sv_ref, dinv_ref):
    su_ref[...] = uvp_ref[0, :, 0:F] + uvp_ref[1, :, 0:F]
    sv_ref[...] = uvp_ref[0, :, F:2 * F] + uvp_ref[1, :, F:2 * F]
    deg = degp_ref[0, :, 0:1] + degp_ref[1, :, 0:1] + 1.0
    dinv_ref[...] = lax.rsqrt(deg)


def _tc_c(w_ref, su_ref, ssvp_ref, dinv_ref, h1_ref, q1_ref):
    h1 = w_ref[...] + su_ref[...] + 2.0 * (ssvp_ref[0] + ssvp_ref[1])
    h1_ref[...] = h1
    q1_ref[...] = dinv_ref[...] * h1


def _tc_conv(mp_ref, h_ref, dinv_ref, b_ref, w_ref, x_ref, hn_ref, qn_ref):
    dinv = dinv_ref[...]
    m = mp_ref[0] + mp_ref[1]
    xk = jax.nn.relu(dinv * m + dinv * dinv * h_ref[...] + b_ref[...])
    rowid = lax.broadcasted_iota(jnp.int32, (NPAD, F), 0)
    xk = jnp.where(rowid < N, xk, 0.0)
    x_ref[...] = xk
    hn = jnp.dot(xk, w_ref[...], preferred_element_type=jnp.float32)
    hn_ref[...] = hn
    qn_ref[...] = dinv * hn


def _tc_f(mp_ref, h3_ref, dinv_ref, b3_ref, x1_ref, x2_ref, batch_ref,
          wl_ref, bl_ref, out_ref):
    dinv = dinv_ref[...]
    m = mp_ref[0] + mp_ref[1]
    x3 = jax.nn.relu(dinv * m + dinv * dinv * h3_ref[...] + b3_ref[...])
    xm = (x1_ref[...] + x2_ref[...] + x3) * (1.0 / 3.0)
    gid = lax.broadcasted_iota(jnp.int32, (G, NPAD), 0)
    onehot = (batch_ref[...][None, :] == gid).astype(jnp.float32)
    sums = jnp.dot(onehot, xm, preferred_element_type=jnp.float32)
    cnt = jnp.sum(onehot, axis=1, keepdims=True)
    pooled = sums / jnp.maximum(cnt, 1.0)
    logits = jnp.dot(pooled, wl_ref[...],
                     preferred_element_type=jnp.float32) + bl_ref[...]
    z = logits - jnp.max(logits, axis=1, keepdims=True)
    ez = jnp.exp(z)
    out_ref[...] = ez / jnp.sum(ez, axis=1, keepdims=True)


def kernel(X, edge_index, L_values, batch, W1, b1, W2, b2, W3, b3, Wl, bl):
    f32 = jnp.float32
    x0 = X[0].astype(f32)
    ei = edge_index.astype(jnp.int32)
    row, col = ei[0], ei[1]
    b0 = batch[0].astype(jnp.int32)

    pad = EPAD - E
    padi = jnp.full((pad,), N, jnp.int32)
    rowp = jnp.concatenate([row, padi])
    colp = jnp.concatenate([col, padi])
    lvp = jnp.concatenate([L_values.astype(f32), jnp.zeros((pad,), f32)])

    x0p = jnp.zeros((NPAD, IN), f32).at[:N].set(x0)
    b0p = jnp.concatenate([b0, jnp.full((NPAD - N,), G, jnp.int32)])
    zeros128 = jnp.zeros((NPAD, 2 * F), f32)
    zeros64 = jnp.zeros((NPAD, F), f32)
    zeros16 = jnp.zeros((NPAD, 16), f32)
    ones16 = jnp.ones((EBLK, 16), f32)

    uv, w = _tc_call(_tc_a, [jax.ShapeDtypeStruct((NPAD, 2 * F), f32),
                             jax.ShapeDtypeStruct((NPAD, F), f32)])(x0p, W1)

    uvp, degp = _pass_uv(uv, colp, rowp, lvp, zeros128, zeros16, ones16)

    su, sv, dinv = _tc_call(_tc_b, [jax.ShapeDtypeStruct((NPAD, F), f32),
                                    jax.ShapeDtypeStruct((NPAD, F), f32),
                                    jax.ShapeDtypeStruct((NPAD, 1), f32)])(
        uvp, degp)

    (ssvp,) = _pass_w64(sv, colp, rowp, lvp, zeros64)

    h1, q1 = _tc_call(_tc_c, [jax.ShapeDtypeStruct((NPAD, F), f32),
                              jax.ShapeDtypeStruct((NPAD, F), f32)])(
        w, su, ssvp, dinv)

    (m1p,) = _pass_u64(q1, rowp, colp, zeros64)
    x1, h2, q2 = _tc_call(_tc_conv, [jax.ShapeDtypeStruct((NPAD, F), f32)] * 3)(
        m1p, h1, dinv, b1, W2)
    (m2p,) = _pass_u64(q2, rowp, colp, zeros64)
    x2, h3, q3 = _tc_call(_tc_conv, [jax.ShapeDtypeStruct((NPAD, F), f32)] * 3)(
        m2p, h2, dinv, b2, W3)
    (m3p,) = _pass_u64(q3, rowp, colp, zeros64)

    out = _tc_call(_tc_f, jax.ShapeDtypeStruct((G, OUT), f32))(
        m3p, h3, dinv, b3, x1, x2, b0p, Wl, bl)
    return out


# speedup vs baseline: 9.1799x; 1.0342x over previous
import functools

import jax
import jax.numpy as jnp
from jax import lax
from jax.experimental import pallas as pl
from jax.experimental.pallas import tpu as pltpu
from jax.experimental.pallas import tpu_sc as plsc

N = 10000
NPAD = 10112
E = 320000
IN = 128
F = 64
OUT = 10
G = 64

EBLK = 128
NC, NS = 2, 16
NW = NC * NS
NBLK = 2560
EPAD = NBLK * EBLK
RSUB = NPAD // NS

_MESH = plsc.VectorSubcoreMesh(core_axis_name="c", subcore_axis_name="s")


def _edge_pass(width, weighted, with_deg, bpw0):
    bpw1 = (NBLK - bpw0 * NS) // NS
    assert bpw0 * NS + bpw1 * NS == NBLK and bpw0 % 2 == 0 and bpw1 % 2 == 0

    out_types = [jax.ShapeDtypeStruct((NC, NPAD, width), jnp.float32)]
    if with_deg:
        out_types.append(jax.ShapeDtypeStruct((NC, NPAD, 16), jnp.float32))

    scratch = [
        pltpu.VMEM((2, EBLK), jnp.int32),
        pltpu.VMEM((2, EBLK), jnp.int32),
        pltpu.VMEM((2, EBLK, width), jnp.float32),
        pltpu.VMEM_SHARED((NPAD, width), jnp.float32),
        pltpu.SemaphoreType.DMA((2,)),
        pltpu.SemaphoreType.DMA((2,)),
    ]
    if weighted:
        scratch.append(pltpu.VMEM((2, EBLK), jnp.float32))
    if with_deg:
        scratch.append(pltpu.VMEM((EBLK, 16), jnp.float32))
        scratch.append(pltpu.VMEM_SHARED((NPAD, 16), jnp.float32))

    def body(*refs):
        i = 0
        table_h = refs[i]; i += 1
        gidx_h = refs[i]; i += 1
        sidx_h = refs[i]; i += 1
        if weighted:
            lv_h = refs[i]; i += 1
        zeros_h = refs[i]; i += 1
        if with_deg:
            zeros16_h = refs[i]; i += 1
            ones_h = refs[i]; i += 1
        out_h = refs[i]; i += 1
        if with_deg:
            dego_h = refs[i]; i += 1
        gidx_v = refs[i]; i += 1
        sidx_v = refs[i]; i += 1
        rows_v = refs[i]; i += 1
        acc = refs[i]; i += 1
        sem_i = refs[i]; i += 1
        sem_g = refs[i]; i += 1
        if weighted:
            lv_v = refs[i]; i += 1
        if with_deg:
            ones_v = refs[i]; i += 1
            dacc = refs[i]; i += 1

        c = lax.axis_index("c")
        s = lax.axis_index("s")
        r0 = s * RSUB
        bpw = lax.select(c == 0, jnp.int32(bpw0), jnp.int32(bpw1))
        base = lax.select(c == 0, s * bpw0, NS * bpw0 + s * bpw1)

        def issue_idx(ib, k):
            e0 = (base + ib) * EBLK
            pltpu.async_copy(gidx_h.at[pl.ds(e0, EBLK)], gidx_v.at[k],
                             sem_i.at[k])
            pltpu.async_copy(sidx_h.at[pl.ds(e0, EBLK)], sidx_v.at[k],
                             sem_i.at[k])
            if weighted:
                pltpu.async_copy(lv_h.at[pl.ds(e0, EBLK)], lv_v.at[k],
                                 sem_i.at[k])

        def wait_idx(ib, k):
            e0 = (base + ib) * EBLK
            pltpu.make_async_copy(gidx_h.at[pl.ds(e0, EBLK)], gidx_v.at[k],
                                  sem_i.at[k]).wait()
            pltpu.make_async_copy(sidx_h.at[pl.ds(e0, EBLK)], sidx_v.at[k],
                                  sem_i.at[k]).wait()
            if weighted:
                pltpu.make_async_copy(lv_h.at[pl.ds(e0, EBLK)], lv_v.at[k],
                                      sem_i.at[k]).wait()

        def issue_gather(k):
            pltpu.async_copy(table_h.at[gidx_v.at[k]], rows_v.at[k],
                             sem_g.at[k])

        def wait_gather(k):
            pltpu.make_async_copy(table_h.at[gidx_v.at[k]], rows_v.at[k],
                                  sem_g.at[k]).wait()

        pltpu.sync_copy(zeros_h.at[pl.ds(r0, RSUB), :],
                        acc.at[pl.ds(r0, RSUB), :])
        if with_deg:
            pltpu.sync_copy(zeros16_h.at[pl.ds(r0, RSUB), :],
                            dacc.at[pl.ds(r0, RSUB), :])
            pltpu.sync_copy(ones_h, ones_v)
        plsc.subcore_barrier()

        issue_idx(0, 0)
        wait_idx(0, 0)
        issue_gather(0)

        @pl.loop(0, bpw // 2)
        def _(g):
            for k in (0, 1):
                ib = g * 2 + k

                @pl.when(ib + 1 < bpw)
                def _():
                    issue_idx(ib + 1, 1 - k)
                wait_gather(k)

                @pl.when(ib + 1 < bpw)
                def _():
                    wait_idx(ib + 1, 1 - k)
                    issue_gather(1 - k)
                if weighted:
                    @pl.loop(0, EBLK // 16)
                    def _(eg):
                        vals = lv_v[k, pl.ds(eg * 16, 16)]
                        for j in range(16):
                            val = vals[j]
                            e = eg * 16 + j
                            for q in range(width // 16):
                                sl = (k, e, pl.ds(q * 16, 16))
                                rows_v[sl] = rows_v[sl] * val
                pltpu.sync_copy(rows_v.at[k], acc.at[sidx_v.at[k]], add=True)
                if with_deg:
                    pltpu.sync_copy(ones_v, dacc.at[gidx_v.at[k]], add=True)

        plsc.subcore_barrier()
        pltpu.sync_copy(acc.at[pl.ds(r0, RSUB), :],
                        out_h.at[c, pl.ds(r0, RSUB), :])
        if with_deg:
            pltpu.sync_copy(dacc.at[pl.ds(r0, RSUB), :],
                            dego_h.at[c, pl.ds(r0, RSUB), :])

    return functools.partial(
        pl.kernel, mesh=_MESH, out_type=out_types, scratch_types=scratch,
        compiler_params=pltpu.CompilerParams(use_tc_tiling_on_sc=False),
    )(body)


_pass_uv = _edge_pass(2 * F, weighted=True, with_deg=True, bpw0=132)
_pass_w64 = _edge_pass(F, weighted=True, with_deg=False, bpw0=124)
_pass_u64 = _edge_pass(F, weighted=False, with_deg=False, bpw0=122)


def _tc_call(fn, out_shapes):
    return pl.pallas_call(fn, out_shape=out_shapes)


def _tc_a(x_ref, w1_ref, uv_ref, w_ref):
    x = x_ref[...]
    w1 = w1_ref[...]
    a, b, c = w1[0:IN], w1[IN:2 * IN], w1[2 * IN:3 * IN]
    uv_ref[:, 0:F] = jnp.dot(x, b, preferred_element_type=jnp.float32)
    uv_ref[:, F:2 * F] = jnp.dot(x, c, preferred_element_type=jnp.float32)
    w_ref[...] = jnp.dot(x, a - c, preferred_element_type=jnp.float32)


def _tc_b(uvp_ref, degp_ref, su_ref, sv_ref, dinv_ref):
    su_ref[...] = uvp_ref[0, :, 0:F] + uvp_ref[1, :, 0:F]
    sv_ref[...] = uvp_ref[0, :, F:2 * F] + uvp_ref[1, :, F:2 * F]
    deg = degp_ref[0, :, 0:1] + degp_ref[1, :, 0:1] + 1.0
    dinv_ref[...] = lax.rsqrt(deg)


def _tc_c(w_ref, su_ref, ssvp_ref, dinv_ref, h1_ref, q1_ref):
    h1 = w_ref[...] + su_ref[...] + 2.0 * (ssvp_ref[0] + ssvp_ref[1])
    h1_ref[...] = h1
    q1_ref[...] = dinv_ref[...] * h1


def _tc_conv(mp_ref, h_ref, dinv_ref, b_ref, w_ref, x_ref, hn_ref, qn_ref):
    dinv = dinv_ref[...]
    m = mp_ref[0] + mp_ref[1]
    xk = jax.nn.relu(dinv * m + dinv * dinv * h_ref[...] + b_ref[...])
    rowid = lax.broadcasted_iota(jnp.int32, (NPAD, F), 0)
    xk = jnp.where(rowid < N, xk, 0.0)
    x_ref[...] = xk
    hn = jnp.dot(xk, w_ref[...], preferred_element_type=jnp.float32)
    hn_ref[...] = hn
    qn_ref[...] = dinv * hn


def _tc_f(mp_ref, h3_ref, dinv_ref, b3_ref, x1_ref, x2_ref, batch_ref,
          wl_ref, bl_ref, out_ref):
    dinv = dinv_ref[...]
    m = mp_ref[0] + mp_ref[1]
    x3 = jax.nn.relu(dinv * m + dinv * dinv * h3_ref[...] + b3_ref[...])
    xm = (x1_ref[...] + x2_ref[...] + x3) * (1.0 / 3.0)
    gid = lax.broadcasted_iota(jnp.int32, (G, NPAD), 0)
    onehot = (batch_ref[...][None, :] == gid).astype(jnp.float32)
    sums = jnp.dot(onehot, xm, preferred_element_type=jnp.float32)
    cnt = jnp.sum(onehot, axis=1, keepdims=True)
    pooled = sums / jnp.maximum(cnt, 1.0)
    logits = jnp.dot(pooled, wl_ref[...],
                     preferred_element_type=jnp.float32) + bl_ref[...]
    z = logits - jnp.max(logits, axis=1, keepdims=True)
    ez = jnp.exp(z)
    out_ref[...] = ez / jnp.sum(ez, axis=1, keepdims=True)


def kernel(X, edge_index, L_values, batch, W1, b1, W2, b2, W3, b3, Wl, bl):
    f32 = jnp.float32
    x0 = X[0].astype(f32)
    ei = edge_index.astype(jnp.int32)
    row, col = ei[0], ei[1]
    b0 = batch[0].astype(jnp.int32)

    pad = EPAD - E
    padi = jnp.full((pad,), N, jnp.int32)
    rowp = jnp.concatenate([row, padi])
    colp = jnp.concatenate([col, padi])
    lvp = jnp.concatenate([L_values.astype(f32), jnp.zeros((pad,), f32)])

    x0p = jnp.zeros((NPAD, IN), f32).at[:N].set(x0)
    b0p = jnp.concatenate([b0, jnp.full((NPAD - N,), G, jnp.int32)])
    zeros128 = jnp.zeros((NPAD, 2 * F), f32)
    zeros64 = jnp.zeros((NPAD, F), f32)
    zeros16 = jnp.zeros((NPAD, 16), f32)
    ones16 = jnp.ones((EBLK, 16), f32)

    uv, w = _tc_call(_tc_a, [jax.ShapeDtypeStruct((NPAD, 2 * F), f32),
                             jax.ShapeDtypeStruct((NPAD, F), f32)])(x0p, W1)

    uvp, degp = _pass_uv(uv, colp, rowp, lvp, zeros128, zeros16, ones16)

    su, sv, dinv = _tc_call(_tc_b, [jax.ShapeDtypeStruct((NPAD, F), f32),
                                    jax.ShapeDtypeStruct((NPAD, F), f32),
                                    jax.ShapeDtypeStruct((NPAD, 1), f32)])(
        uvp, degp)

    (ssvp,) = _pass_w64(sv, colp, rowp, lvp, zeros64)

    h1, q1 = _tc_call(_tc_c, [jax.ShapeDtypeStruct((NPAD, F), f32),
                              jax.ShapeDtypeStruct((NPAD, F), f32)])(
        w, su, ssvp, dinv)

    (m1p,) = _pass_u64(q1, rowp, colp, zeros64)
    x1, h2, q2 = _tc_call(_tc_conv, [jax.ShapeDtypeStruct((NPAD, F), f32)] * 3)(
        m1p, h1, dinv, b1, W2)
    (m2p,) = _pass_u64(q2, rowp, colp, zeros64)
    x2, h3, q3 = _tc_call(_tc_conv, [jax.ShapeDtypeStruct((NPAD, F), f32)] * 3)(
        m2p, h2, dinv, b2, W3)
    (m3p,) = _pass_u64(q3, rowp, colp, zeros64)

    out = _tc_call(_tc_f, jax.ShapeDtypeStruct((G, OUT), f32))(
        m3p, h3, dinv, b3, x1, x2, b0p, Wl, bl)
    return out
